# baseline (speedup 1.0000x reference)
import functools

import jax
import jax.numpy as jnp
from jax import lax
from jax.experimental import pallas as pl
from jax.experimental.pallas import tpu as pltpu
from jax.experimental.pallas import tpu_sc as plsc

NUM_BUCKETS = 1000000
HASH_OFFSET = 13

B = 16384
LANES = 16
NUM_WORKERS = 32
PER_W = B // NUM_WORKERS
GATHER_CHUNK = 128
NUM_CHUNKS = PER_W // GATHER_CHUNK

_LN2_HI = 0.693359375
_LN2_LO = -2.12194440e-4
_SQRT2 = 1.41421356237
_POLY = (7.0376836292e-2, -1.1514610310e-1, 1.1676998740e-1,
         -1.2420140846e-1, 1.4249322787e-1, -1.6668057665e-1,
         2.0000714765e-1, -2.4999993993e-1, 3.3333331174e-1)


def _neg_log16(x):
    bits = lax.bitcast_convert_type(x, jnp.int32)
    e = lax.shift_right_logical(bits, 23) - 127
    m = lax.bitcast_convert_type(
        (bits & 0x007FFFFF) | 0x3F800000, jnp.float32)
    big = m > _SQRT2
    e = jnp.where(big, e + 1, e)
    m = jnp.where(big, m * 0.5, m)
    t = m - 1.0
    z = t * t
    y = jnp.full((LANES,), _POLY[0], jnp.float32)
    for c in _POLY[1:]:
        y = y * t + c
    y = y * t * z
    ef = e.astype(jnp.float32)
    y = y + ef * _LN2_LO
    y = y - 0.5 * z
    return -(t + y + ef * _LN2_HI)


def _body(products_hbm, b_hbm, out_hbm, prod_v, idx_v, vals_v,
          sem_in, sem_g0, sem_g1, sem_g2, sem_g3, sem_out):
    gather_sems = (sem_g0, sem_g1, sem_g2, sem_g3)
    wid = lax.axis_index("s") * 2 + lax.axis_index("c")
    base = wid * PER_W
    pltpu.async_copy(products_hbm.at[pl.ds(base, PER_W)], prod_v, sem_in).wait()

    gathers = []
    for c in range(NUM_CHUNKS):
        @plsc.parallel_loop(c * GATHER_CHUNK, (c + 1) * GATHER_CHUNK,
                            step=LANES, unroll=4)
        def _hash(i):
            p = prod_v[pl.ds(i, LANES)]
            h = p + HASH_OFFSET
            idx_v[pl.ds(i, LANES)] = jnp.where(
                h >= NUM_BUCKETS, h - NUM_BUCKETS, h)

        gathers.append(pltpu.async_copy(
            b_hbm.at[idx_v.at[pl.ds(c * GATHER_CHUNK, GATHER_CHUNK)]],
            vals_v.at[pl.ds(c * GATHER_CHUNK, GATHER_CHUNK)],
            gather_sems[c],
        ))

    outs = []
    for c in range(NUM_CHUNKS):
        gathers[c].wait()

        @plsc.parallel_loop(c * GATHER_CHUNK, (c + 1) * GATHER_CHUNK,
                            step=LANES, unroll=4)
        def _nlog(i):
            vals_v[pl.ds(i, LANES)] = _neg_log16(vals_v[pl.ds(i, LANES)])

        outs.append(pltpu.async_copy(
            vals_v.at[pl.ds(c * GATHER_CHUNK, GATHER_CHUNK)],
            out_hbm.at[pl.ds(base + c * GATHER_CHUNK, GATHER_CHUNK)],
            sem_out,
        ))
    for cp in outs:
        cp.wait()


@jax.jit
def kernel(products, b):
    mesh = plsc.VectorSubcoreMesh(core_axis_name="c", subcore_axis_name="s")
    run = functools.partial(
        pl.kernel,
        mesh=mesh,
        out_type=jax.ShapeDtypeStruct((B,), jnp.float32),
        scratch_types=[
            pltpu.VMEM((PER_W,), jnp.int32),
            pltpu.VMEM((PER_W,), jnp.int32),
            pltpu.VMEM((PER_W,), jnp.float32),
        ] + [pltpu.SemaphoreType.DMA] * 6,
    )(_body)
    return run(products, b)


# speedup vs baseline: 1.0177x; 1.0177x over previous
import functools

import jax
import jax.numpy as jnp
from jax import lax
from jax.experimental import pallas as pl
from jax.experimental.pallas import tpu as pltpu
from jax.experimental.pallas import tpu_sc as plsc

NUM_BUCKETS = 1000000
HASH_OFFSET = 13

B = 16384
LANES = 16
NUM_WORKERS = 32
PER_W = B // NUM_WORKERS
GATHER_CHUNK = 128
NUM_CHUNKS = PER_W // GATHER_CHUNK

_LN2_HI = 0.693359375
_LN2_LO = -2.12194440e-4
_SQRT2 = 1.41421356237
_POLY = (7.0376836292e-2, -1.1514610310e-1, 1.1676998740e-1,
         -1.2420140846e-1, 1.4249322787e-1, -1.6668057665e-1,
         2.0000714765e-1, -2.4999993993e-1, 3.3333331174e-1)


def _neg_log16(x):
    bits = lax.bitcast_convert_type(x, jnp.int32)
    e = lax.shift_right_logical(bits, 23) - 127
    m = lax.bitcast_convert_type(
        (bits & 0x007FFFFF) | 0x3F800000, jnp.float32)
    big = m > _SQRT2
    e = jnp.where(big, e + 1, e)
    m = jnp.where(big, m * 0.5, m)
    t = m - 1.0
    z = t * t
    y = jnp.full((LANES,), _POLY[0], jnp.float32)
    for c in _POLY[1:]:
        y = y * t + c
    y = y * t * z
    ef = e.astype(jnp.float32)
    y = y + ef * _LN2_LO
    y = y - 0.5 * z
    return -(t + y + ef * _LN2_HI)


HALF = PER_W // 2


def _body(products_hbm, b_hbm, out_hbm, prod_v, idx_v, vals_v,
          sem_in, sem_g0, sem_g1, sem_out):
    wid = lax.axis_index("s") * 2 + lax.axis_index("c")
    base = wid * PER_W
    pltpu.async_copy(products_hbm.at[pl.ds(base, PER_W)], prod_v, sem_in).wait()

    @plsc.parallel_loop(0, PER_W, step=LANES, unroll=4)
    def _hash(i):
        p = prod_v[pl.ds(i, LANES)]
        h = p + HASH_OFFSET
        idx_v[pl.ds(i, LANES)] = jnp.where(h >= NUM_BUCKETS, h - NUM_BUCKETS, h)

    gathers = []
    for c in range(NUM_CHUNKS):
        gathers.append(pltpu.async_copy(
            b_hbm.at[idx_v.at[pl.ds(c * GATHER_CHUNK, GATHER_CHUNK)]],
            vals_v.at[pl.ds(c * GATHER_CHUNK, GATHER_CHUNK)],
            sem_g0 if c < NUM_CHUNKS // 2 else sem_g1,
        ))

    outs = []
    for half, lo in ((0, 0), (1, HALF)):
        for cp in gathers[half * (NUM_CHUNKS // 2):(half + 1) * (NUM_CHUNKS // 2)]:
            cp.wait()

        @plsc.parallel_loop(lo, lo + HALF, step=LANES, unroll=4)
        def _nlog(i):
            vals_v[pl.ds(i, LANES)] = _neg_log16(vals_v[pl.ds(i, LANES)])

        outs.append(pltpu.async_copy(
            vals_v.at[pl.ds(lo, HALF)],
            out_hbm.at[pl.ds(base + lo, HALF)],
            sem_out,
        ))
    for cp in outs:
        cp.wait()


@jax.jit
def kernel(products, b):
    mesh = plsc.VectorSubcoreMesh(core_axis_name="c", subcore_axis_name="s")
    run = functools.partial(
        pl.kernel,
        mesh=mesh,
        out_type=jax.ShapeDtypeStruct((B,), jnp.float32),
        scratch_types=[
            pltpu.VMEM((PER_W,), jnp.int32),
            pltpu.VMEM((PER_W,), jnp.int32),
            pltpu.VMEM((PER_W,), jnp.float32),
        ] + [pltpu.SemaphoreType.DMA] * 4,
    )(_body)
    return run(products, b)


# speedup vs baseline: 1.0241x; 1.0063x over previous
import functools

import jax
import jax.numpy as jnp
from jax import lax
from jax.experimental import pallas as pl
from jax.experimental.pallas import tpu as pltpu
from jax.experimental.pallas import tpu_sc as plsc

NUM_BUCKETS = 1000000
HASH_OFFSET = 13

B = 16384
LANES = 16
NUM_WORKERS = 32
PER_W = B // NUM_WORKERS
GATHER_CHUNK = 128
NUM_CHUNKS = PER_W // GATHER_CHUNK

_LN2_HI = 0.693359375
_LN2_LO = -2.12194440e-4
_SQRT2 = 1.41421356237
_POLY = (7.0376836292e-2, -1.1514610310e-1, 1.1676998740e-1,
         -1.2420140846e-1, 1.4249322787e-1, -1.6668057665e-1,
         2.0000714765e-1, -2.4999993993e-1, 3.3333331174e-1)


def _neg_log16(x):
    bits = lax.bitcast_convert_type(x, jnp.int32)
    e = lax.shift_right_logical(bits, 23) - 127
    m = lax.bitcast_convert_type(
        (bits & 0x007FFFFF) | 0x3F800000, jnp.float32)
    big = m > _SQRT2
    e = jnp.where(big, e + 1, e)
    m = jnp.where(big, m * 0.5, m)
    t = m - 1.0
    z = t * t
    y = jnp.full((LANES,), _POLY[0], jnp.float32)
    for c in _POLY[1:]:
        y = y * t + c
    y = y * t * z
    ef = e.astype(jnp.float32)
    y = y + ef * _LN2_LO
    y = y - 0.5 * z
    return -(t + y + ef * _LN2_HI)


def _body(products_hbm, b_hbm, out_hbm, prod_v, idx_v, vals_v, sem):
    wid = lax.axis_index("s") * 2 + lax.axis_index("c")
    base = wid * PER_W
    pltpu.sync_copy(products_hbm.at[pl.ds(base, PER_W)], prod_v)

    @plsc.parallel_loop(0, PER_W, step=LANES, unroll=8)
    def _hash(i):
        p = prod_v[pl.ds(i, LANES)]
        h = p + HASH_OFFSET
        idx_v[pl.ds(i, LANES)] = jnp.where(h >= NUM_BUCKETS, h - NUM_BUCKETS, h)

    copies = [
        pltpu.async_copy(
            b_hbm.at[idx_v.at[pl.ds(c * GATHER_CHUNK, GATHER_CHUNK)]],
            vals_v.at[pl.ds(c * GATHER_CHUNK, GATHER_CHUNK)],
            sem,
        )
        for c in range(NUM_CHUNKS)
    ]
    for cp in copies:
        cp.wait()

    @plsc.parallel_loop(0, PER_W, step=LANES, unroll=8)
    def _nlog(i):
        vals_v[pl.ds(i, LANES)] = _neg_log16(vals_v[pl.ds(i, LANES)])

    pltpu.sync_copy(vals_v, out_hbm.at[pl.ds(base, PER_W)])


@jax.jit
def kernel(products, b):
    mesh = plsc.VectorSubcoreMesh(core_axis_name="c", subcore_axis_name="s")
    run = functools.partial(
        pl.kernel,
        mesh=mesh,
        out_type=jax.ShapeDtypeStruct((B,), jnp.float32),
        scratch_types=[
            pltpu.VMEM((PER_W,), jnp.int32),
            pltpu.VMEM((PER_W,), jnp.int32),
            pltpu.VMEM((PER_W,), jnp.float32),
            pltpu.SemaphoreType.DMA,
        ],
    )(_body)
    return run(products, b)


# speedup vs baseline: 1.0257x; 1.0016x over previous
import functools

import jax
import jax.numpy as jnp
from jax import lax
from jax.experimental import pallas as pl
from jax.experimental.pallas import tpu as pltpu
from jax.experimental.pallas import tpu_sc as plsc

NUM_BUCKETS = 1000000
HASH_OFFSET = 13

B = 16384
LANES = 16
NUM_WORKERS = 32
PER_W = B // NUM_WORKERS
GATHER_CHUNK = 128
NUM_CHUNKS = PER_W // GATHER_CHUNK

_LN2_HI = 0.693359375
_LN2_LO = -2.12194440e-4
_SQRT2 = 1.41421356237
_POLY = (7.0376836292e-2, -1.1514610310e-1, 1.1676998740e-1,
         -1.2420140846e-1, 1.4249322787e-1, -1.6668057665e-1,
         2.0000714765e-1, -2.4999993993e-1, 3.3333331174e-1)


def _neg_log16(x):
    bits = lax.bitcast_convert_type(x, jnp.int32)
    e = lax.shift_right_logical(bits, 23) - 127
    m = lax.bitcast_convert_type(
        (bits & 0x007FFFFF) | 0x3F800000, jnp.float32)
    big = m > _SQRT2
    e = jnp.where(big, e + 1, e)
    m = jnp.where(big, m * 0.5, m)
    t = m - 1.0
    z = t * t
    y = jnp.full((LANES,), _POLY[0], jnp.float32)
    for c in _POLY[1:]:
        y = y * t + c
    y = y * t * z
    ef = e.astype(jnp.float32)
    y = y + ef * _LN2_LO
    y = y - 0.5 * z
    return -(t + y + ef * _LN2_HI)


def _body(products_hbm, b_hbm, out_hbm, prod_v, idx_v, vals_v, sem):
    wid = lax.axis_index("s") * 2 + lax.axis_index("c")
    base = wid * PER_W
    pltpu.sync_copy(products_hbm.at[pl.ds(base, PER_W)], prod_v)

    @plsc.parallel_loop(0, PER_W, step=LANES, unroll=4)
    def _hash(i):
        p = prod_v[pl.ds(i, LANES)]
        h = p + HASH_OFFSET
        idx_v[pl.ds(i, LANES)] = jnp.where(h >= NUM_BUCKETS, h - NUM_BUCKETS, h)

    pltpu.async_copy(b_hbm.at[idx_v], vals_v, sem).wait()

    @plsc.parallel_loop(0, PER_W, step=LANES, unroll=4)
    def _nlog(i):
        vals_v[pl.ds(i, LANES)] = _neg_log16(vals_v[pl.ds(i, LANES)])

    pltpu.sync_copy(vals_v, out_hbm.at[pl.ds(base, PER_W)])


@jax.jit
def kernel(products, b):
    mesh = plsc.VectorSubcoreMesh(core_axis_name="c", subcore_axis_name="s")
    run = functools.partial(
        pl.kernel,
        mesh=mesh,
        out_type=jax.ShapeDtypeStruct((B,), jnp.float32),
        scratch_types=[
            pltpu.VMEM((PER_W,), jnp.int32),
            pltpu.VMEM((PER_W,), jnp.int32),
            pltpu.VMEM((PER_W,), jnp.float32),
            pltpu.SemaphoreType.DMA,
        ],
    )(_body)
    return run(products, b)


# speedup vs baseline: 1.0375x; 1.0115x over previous
import functools

import jax
import jax.numpy as jnp
from jax import lax
from jax.experimental import pallas as pl
from jax.experimental.pallas import tpu as pltpu
from jax.experimental.pallas import tpu_sc as plsc

NUM_BUCKETS = 1000000
HASH_OFFSET = 13

B = 16384
LANES = 16
NUM_WORKERS = 32
PER_W = B // NUM_WORKERS
GATHER_CHUNK = 128
NUM_CHUNKS = PER_W // GATHER_CHUNK

_LN2_HI = 0.693359375
_LN2_LO = -2.12194440e-4
_SQRT2 = 1.41421356237
_POLY = (7.0376836292e-2, -1.1514610310e-1, 1.1676998740e-1,
         -1.2420140846e-1, 1.4249322787e-1, -1.6668057665e-1,
         2.0000714765e-1, -2.4999993993e-1, 3.3333331174e-1)


def _neg_log16(x):
    bits = lax.bitcast_convert_type(x, jnp.int32)
    e = lax.shift_right_logical(bits, 23) - 127
    m = lax.bitcast_convert_type(
        (bits & 0x007FFFFF) | 0x3F800000, jnp.float32)
    big = m > _SQRT2
    e = jnp.where(big, e + 1, e)
    m = jnp.where(big, m * 0.5, m)
    t = m - 1.0
    z = t * t
    y = jnp.full((LANES,), _POLY[0], jnp.float32)
    for c in _POLY[1:]:
        y = y * t + c
    y = y * t * z
    ef = e.astype(jnp.float32)
    y = y + ef * _LN2_LO
    y = y - 0.5 * z
    return -(t + y + ef * _LN2_HI)


def _body(products_hbm, b_hbm, out_hbm, prod_v, idx_v, vals_v, sem):
    wid = lax.axis_index("s") * 2 + lax.axis_index("c")
    base = wid * PER_W
    pltpu.sync_copy(products_hbm.at[pl.ds(base, PER_W)], prod_v)

    @plsc.parallel_loop(0, PER_W, step=LANES, unroll=4)
    def _hash(i):
        p = prod_v[pl.ds(i, LANES)]
        h = p + HASH_OFFSET
        idx_v[pl.ds(i, LANES)] = jnp.where(h >= NUM_BUCKETS, h - NUM_BUCKETS, h)

    copies = [
        pltpu.async_copy(
            b_hbm.at[idx_v.at[pl.ds(c * GATHER_CHUNK, GATHER_CHUNK)]],
            vals_v.at[pl.ds(c * GATHER_CHUNK, GATHER_CHUNK)],
            sem,
        )
        for c in range(NUM_CHUNKS)
    ]
    for cp in copies:
        cp.wait()

    @plsc.parallel_loop(0, PER_W, step=LANES, unroll=4)
    def _nlog(i):
        vals_v[pl.ds(i, LANES)] = _neg_log16(vals_v[pl.ds(i, LANES)])

    pltpu.sync_copy(vals_v, out_hbm.at[pl.ds(base, PER_W)])


@jax.jit
def kernel(products, b):
    mesh = plsc.VectorSubcoreMesh(core_axis_name="c", subcore_axis_name="s")
    run = functools.partial(
        pl.kernel,
        mesh=mesh,
        out_type=jax.ShapeDtypeStruct((B,), jnp.float32),
        scratch_types=[
            pltpu.VMEM((PER_W,), jnp.int32),
            pltpu.VMEM((PER_W,), jnp.int32),
            pltpu.VMEM((PER_W,), jnp.float32),
            pltpu.SemaphoreType.DMA,
        ],
    )(_body)
    return run(products, b)


# speedup vs baseline: 1.0490x; 1.0111x over previous
import functools

import jax
import jax.numpy as jnp
from jax import lax
from jax.experimental import pallas as pl
from jax.experimental.pallas import tpu as pltpu
from jax.experimental.pallas import tpu_sc as plsc

NUM_BUCKETS = 1000000
HASH_OFFSET = 13

B = 16384
LANES = 16
NUM_WORKERS = 16
PER_W = B // NUM_WORKERS
GATHER_CHUNK = 128
NUM_CHUNKS = PER_W // GATHER_CHUNK

_LN2_HI = 0.693359375
_LN2_LO = -2.12194440e-4
_SQRT2 = 1.41421356237
_POLY = (7.0376836292e-2, -1.1514610310e-1, 1.1676998740e-1,
         -1.2420140846e-1, 1.4249322787e-1, -1.6668057665e-1,
         2.0000714765e-1, -2.4999993993e-1, 3.3333331174e-1)


def _neg_log16(x):
    bits = lax.bitcast_convert_type(x, jnp.int32)
    e = lax.shift_right_logical(bits, 23) - 127
    m = lax.bitcast_convert_type(
        (bits & 0x007FFFFF) | 0x3F800000, jnp.float32)
    big = m > _SQRT2
    e = jnp.where(big, e + 1, e)
    m = jnp.where(big, m * 0.5, m)
    t = m - 1.0
    z = t * t
    y = jnp.full((LANES,), _POLY[0], jnp.float32)
    for c in _POLY[1:]:
        y = y * t + c
    y = y * t * z
    ef = e.astype(jnp.float32)
    y = y + ef * _LN2_LO
    y = y - 0.5 * z
    return -(t + y + ef * _LN2_HI)


def _body(products_hbm, b_hbm, out_hbm, prod_v, idx_v, vals_v, sem):
    wid = lax.axis_index("s")
    base = wid * PER_W
    pltpu.sync_copy(products_hbm.at[pl.ds(base, PER_W)], prod_v)

    @plsc.parallel_loop(0, PER_W, step=LANES, unroll=4)
    def _hash(i):
        p = prod_v[pl.ds(i, LANES)]
        h = p + HASH_OFFSET
        idx_v[pl.ds(i, LANES)] = jnp.where(h >= NUM_BUCKETS, h - NUM_BUCKETS, h)

    copies = [
        pltpu.async_copy(
            b_hbm.at[idx_v.at[pl.ds(c * GATHER_CHUNK, GATHER_CHUNK)]],
            vals_v.at[pl.ds(c * GATHER_CHUNK, GATHER_CHUNK)],
            sem,
        )
        for c in range(NUM_CHUNKS)
    ]
    for cp in copies:
        cp.wait()

    @plsc.parallel_loop(0, PER_W, step=LANES, unroll=4)
    def _nlog(i):
        vals_v[pl.ds(i, LANES)] = _neg_log16(vals_v[pl.ds(i, LANES)])

    pltpu.sync_copy(vals_v, out_hbm.at[pl.ds(base, PER_W)])


@jax.jit
def kernel(products, b):
    mesh = plsc.VectorSubcoreMesh(core_axis_name="c", subcore_axis_name="s", num_cores=1)
    run = functools.partial(
        pl.kernel,
        mesh=mesh,
        out_type=jax.ShapeDtypeStruct((B,), jnp.float32),
        scratch_types=[
            pltpu.VMEM((PER_W,), jnp.int32),
            pltpu.VMEM((PER_W,), jnp.int32),
            pltpu.VMEM((PER_W,), jnp.float32),
            pltpu.SemaphoreType.DMA,
        ],
    )(_body)
    return run(products, b)
